# Initial kernel scaffold; baseline (speedup 1.0000x reference)
import jax
import jax.numpy as jnp
from jax.experimental import pallas as pl


def kernel(img1, img2, f1, f2, params):
    raise NotImplementedError("write your pallas kernel here")


# speedup vs baseline: 2.2230x; 2.2230x over previous
import functools

import jax
import jax.numpy as jnp
from jax import lax
from jax.experimental import pallas as pl
from jax.experimental.pallas import tpu as pltpu

N = 1024
KNN = 3
NEG = -3.0e38
HI = lax.Precision.HIGHEST
F32 = jnp.float32


def _mm_bias_lrelu_k(w_ref, b_ref, p_ref, o_ref, *, slope):
    y = jnp.dot(w_ref[...], p_ref[0], preferred_element_type=F32, precision=HI) + b_ref[...]
    o_ref[0] = jnp.where(y >= 0, y, slope * y)


def _im2col(x, ksize, stride, pad):
    B, C, H, W = x.shape
    if pad:
        x = jnp.pad(x, ((0, 0), (0, 0), (pad, pad), (pad, pad)))
    Ho = (H + 2 * pad - ksize) // stride + 1
    Wo = (W + 2 * pad - ksize) // stride + 1
    cols = []
    for di in range(ksize):
        for dj in range(ksize):
            cols.append(x[:, :, di:di + stride * (Ho - 1) + 1:stride,
                          dj:dj + stride * (Wo - 1) + 1:stride])
    p = jnp.stack(cols, axis=2)
    return p.reshape(B, C * ksize * ksize, Ho * Wo)


def _conv(x, w, b, ksize, stride, pad, slope):
    B = x.shape[0]
    O = w.shape[0]
    Ho = (x.shape[2] + 2 * pad - ksize) // stride + 1
    Wo = (x.shape[3] + 2 * pad - ksize) // stride + 1
    p = _im2col(x, ksize, stride, pad)
    Kd, Np = p.shape[1], p.shape[2]
    wf = w.reshape(O, Kd)
    out = pl.pallas_call(
        functools.partial(_mm_bias_lrelu_k, slope=slope),
        grid=(B,),
        in_specs=[pl.BlockSpec((O, Kd), lambda i: (0, 0)),
                  pl.BlockSpec((O, 1), lambda i: (0, 0)),
                  pl.BlockSpec((1, Kd, Np), lambda i: (i, 0, 0))],
        out_specs=pl.BlockSpec((1, O, Np), lambda i: (i, 0, 0)),
        out_shape=jax.ShapeDtypeStruct((B, O, Np), F32),
        compiler_params=pltpu.CompilerParams(dimension_semantics=("parallel",)),
    )(wf, b[:, None], p)
    return out.reshape(B, O, Ho, Wo)


def _ec_k(x_ref, w1_ref, wd_ref, m_ref, s1_ref, s2_ref):
    i = pl.program_id(1)
    x = x_ref[0]
    w1 = w1_ref[...]
    wd = wd_ref[...]
    xtx = lax.dot_general(x, x, (((0,), (0,)), ((), ())),
                          preferred_element_type=F32, precision=HI)
    xx = jnp.sum(x * x, axis=0, keepdims=True)
    s = 2.0 * xtx - xx
    iota = lax.broadcasted_iota(jnp.int32, (N, N), 1)
    yc = jnp.dot(wd, x, preferred_element_type=F32, precision=HI)
    acc_max = acc_s1 = acc_s2 = None
    for k in range(KNN):
        v = jnp.max(s, axis=1, keepdims=True)
        t = jnp.where(s == v, iota, N)
        idx = jnp.min(t, axis=1, keepdims=True)
        sel = iota == idx
        oh = sel.astype(F32)
        xg = lax.dot_general(x, oh, (((1,), (1,)), ((), ())),
                             preferred_element_type=F32, precision=HI)
        out = jnp.dot(w1, xg, preferred_element_type=F32, precision=HI) + yc
        if k == 0:
            acc_max, acc_s1, acc_s2 = out, out, out * out
        else:
            acc_max = jnp.maximum(acc_max, out)
            acc_s1 = acc_s1 + out
            acc_s2 = acc_s2 + out * out
        if k < KNN - 1:
            s = jnp.where(sel, NEG, s)
    m_ref[0] = acc_max
    sum1 = jnp.sum(acc_s1, axis=1, keepdims=True)
    sum2 = jnp.sum(acc_s2, axis=1, keepdims=True)

    @pl.when(i == 0)
    def _():
        s1_ref[0] = sum1
        s2_ref[0] = sum2

    @pl.when(i != 0)
    def _():
        s1_ref[0] += sum1
        s2_ref[0] += sum2


def _aff_k(x_ref, sc_ref, sh_ref, o_ref, *, slope):
    z = x_ref[0] * sc_ref[0] + sh_ref[0]
    o_ref[0] = jnp.where(z >= 0, z, slope * z)


def _affine_lrelu(x, scale, shift, slope):
    B, O, n = x.shape
    half = B // 2
    return pl.pallas_call(
        functools.partial(_aff_k, slope=slope),
        grid=(B,),
        in_specs=[pl.BlockSpec((1, O, n), lambda b: (b, 0, 0)),
                  pl.BlockSpec((1, O, 1), lambda b: (b // half, 0, 0)),
                  pl.BlockSpec((1, O, 1), lambda b: (b // half, 0, 0))],
        out_specs=pl.BlockSpec((1, O, n), lambda b: (b, 0, 0)),
        out_shape=jax.ShapeDtypeStruct((B, O, n), F32),
        compiler_params=pltpu.CompilerParams(dimension_semantics=("parallel",)),
    )(x, scale, shift)


def _bn_affine(s1, s2, cnt, gamma, beta):
    mean = s1 / cnt
    var = s2 / cnt - mean * mean
    scale = gamma[None, :, None] / jnp.sqrt(var + 1e-5)
    shift = beta[None, :, None] - mean * scale
    return scale, shift


def _edgeconv(x, w, gamma, beta):
    B, C, n = x.shape
    O = w.shape[0]
    half = B // 2
    w1 = w[:, :C]
    wd = w[:, C:] - w1
    m, s1, s2 = pl.pallas_call(
        _ec_k,
        grid=(2, half),
        in_specs=[pl.BlockSpec((1, C, n), lambda h, i: (h * half + i, 0, 0)),
                  pl.BlockSpec((O, C), lambda h, i: (0, 0)),
                  pl.BlockSpec((O, C), lambda h, i: (0, 0))],
        out_specs=[pl.BlockSpec((1, O, n), lambda h, i: (h * half + i, 0, 0)),
                   pl.BlockSpec((1, O, 1), lambda h, i: (h, 0, 0)),
                   pl.BlockSpec((1, O, 1), lambda h, i: (h, 0, 0))],
        out_shape=[jax.ShapeDtypeStruct((B, O, n), F32),
                   jax.ShapeDtypeStruct((2, O, 1), F32),
                   jax.ShapeDtypeStruct((2, O, 1), F32)],
        compiler_params=pltpu.CompilerParams(
            dimension_semantics=("parallel", "arbitrary")),
    )(x, w1, wd)
    scale, shift = _bn_affine(s1, s2, half * n * KNN, gamma, beta)
    return _affine_lrelu(m, scale, shift, 0.2)


def _c5g_k(g_ref, w_ref, y_ref, s1_ref, s2_ref):
    i = pl.program_id(1)
    y = jnp.dot(w_ref[...], g_ref[0], preferred_element_type=F32, precision=HI)
    y_ref[0] = y
    sum1 = jnp.sum(y, axis=1, keepdims=True)
    sum2 = jnp.sum(y * y, axis=1, keepdims=True)

    @pl.when(i == 0)
    def _():
        s1_ref[0] = sum1
        s2_ref[0] = sum2

    @pl.when(i != 0)
    def _():
        s1_ref[0] += sum1
        s2_ref[0] += sum2


def _affred_k(y_ref, sc_ref, sh_ref, mx_ref, av_ref):
    z = y_ref[0] * sc_ref[0] + sh_ref[0]
    z = jnp.where(z >= 0, z, 0.2 * z)
    mx_ref[0] = jnp.max(z, axis=1, keepdims=True)
    av_ref[0] = jnp.sum(z, axis=1, keepdims=True) * (1.0 / N)


def _c5g_pool(g, w, gamma, beta):
    B, C, n = g.shape
    O = w.shape[0]
    half = B // 2
    y, s1, s2 = pl.pallas_call(
        _c5g_k,
        grid=(2, half),
        in_specs=[pl.BlockSpec((1, C, n), lambda h, i: (h * half + i, 0, 0)),
                  pl.BlockSpec((O, C), lambda h, i: (0, 0))],
        out_specs=[pl.BlockSpec((1, O, n), lambda h, i: (h * half + i, 0, 0)),
                   pl.BlockSpec((1, O, 1), lambda h, i: (h, 0, 0)),
                   pl.BlockSpec((1, O, 1), lambda h, i: (h, 0, 0))],
        out_shape=[jax.ShapeDtypeStruct((B, O, n), F32),
                   jax.ShapeDtypeStruct((2, O, 1), F32),
                   jax.ShapeDtypeStruct((2, O, 1), F32)],
        compiler_params=pltpu.CompilerParams(
            dimension_semantics=("parallel", "arbitrary")),
    )(g, w)
    scale, shift = _bn_affine(s1, s2, half * n, gamma, beta)
    gmax, gavg = pl.pallas_call(
        _affred_k,
        grid=(B,),
        in_specs=[pl.BlockSpec((1, O, n), lambda b: (b, 0, 0)),
                  pl.BlockSpec((1, O, 1), lambda b: (b // half, 0, 0)),
                  pl.BlockSpec((1, O, 1), lambda b: (b // half, 0, 0))],
        out_specs=[pl.BlockSpec((1, O, 1), lambda b: (b, 0, 0)),
                   pl.BlockSpec((1, O, 1), lambda b: (b, 0, 0))],
        out_shape=[jax.ShapeDtypeStruct((B, O, 1), F32),
                   jax.ShapeDtypeStruct((B, O, 1), F32)],
        compiler_params=pltpu.CompilerParams(dimension_semantics=("parallel",)),
    )(y, scale, shift)
    return jnp.concatenate([gmax[:, :, 0], gavg[:, :, 0]], axis=1)


def _linT_k(w_ref, a_ref, b_ref, o_ref):
    o_ref[...] = (jnp.dot(w_ref[...], a_ref[...], preferred_element_type=F32, precision=HI)
                  + b_ref[...])


def _lin_T(w, aT, b, row_tile):
    R, K = w.shape
    B = aT.shape[1]
    nt = R // row_tile
    return pl.pallas_call(
        _linT_k,
        grid=(nt,),
        in_specs=[pl.BlockSpec((row_tile, K), lambda t: (t, 0)),
                  pl.BlockSpec((K, B), lambda t: (0, 0)),
                  pl.BlockSpec((row_tile, 1), lambda t: (t, 0))],
        out_specs=pl.BlockSpec((row_tile, B), lambda t: (t, 0)),
        out_shape=jax.ShapeDtypeStruct((R, B), F32),
        compiler_params=pltpu.CompilerParams(dimension_semantics=("parallel",)),
    )(w, aT, b[:, None])


def _lin1_k(w_ref, a_ref, b_ref, o_ref):
    k = pl.program_id(1)
    part = jnp.dot(w_ref[...], a_ref[...], preferred_element_type=F32, precision=HI)

    @pl.when(k == 0)
    def _():
        o_ref[...] = part + b_ref[...]

    @pl.when(k != 0)
    def _():
        o_ref[...] += part


def _lin1_T(w, aT, b, k_tile):
    R, K = w.shape
    B = aT.shape[1]
    nk = K // k_tile
    rt = R // 2
    return pl.pallas_call(
        _lin1_k,
        grid=(2, nk),
        in_specs=[pl.BlockSpec((rt, k_tile), lambda r, k: (r, k)),
                  pl.BlockSpec((k_tile, B), lambda r, k: (k, 0)),
                  pl.BlockSpec((rt, 1), lambda r, k: (r, 0))],
        out_specs=pl.BlockSpec((rt, B), lambda r, k: (r, 0)),
        out_shape=jax.ShapeDtypeStruct((R, B), F32),
        compiler_params=pltpu.CompilerParams(
            dimension_semantics=("parallel", "arbitrary")),
    )(w, aT, b[:, None])


def _head_k(x_ref, ew_ref, eb_ref, cl_ref, emb_ref, dis_ref, q_ref, sim_ref):
    xT = x_ref[...]
    embT = (jnp.dot(ew_ref[...], xT, preferred_element_type=F32, precision=HI)
            + eb_ref[...])
    clu = cl_ref[...]
    cc = jnp.sum(clu * clu, axis=1, keepdims=True)
    ee = jnp.sum(embT * embT, axis=0, keepdims=True)
    cross = jnp.dot(clu, embT, preferred_element_type=F32, precision=HI)
    dis = cc - 2.0 * cross + ee
    q = 1.0 / (1.0 + dis)
    q = q / jnp.sum(q, axis=0, keepdims=True)
    d = embT[:, 0:8] - embT[:, 8:16] + 1e-6
    sim = jnp.sqrt(jnp.sum(d * d, axis=0, keepdims=True))
    emb_ref[...] = embT
    dis_ref[...] = dis
    q_ref[...] = q
    sim_ref[...] = sim


def _head(xT, emb_w, emb_b, clu_w):
    B = xT.shape[1]
    nc = clu_w.shape[0]
    embT, disT, qT, sim = pl.pallas_call(
        _head_k,
        in_specs=[pl.BlockSpec(xT.shape, lambda: (0, 0)),
                  pl.BlockSpec(emb_w.shape, lambda: (0, 0)),
                  pl.BlockSpec((emb_w.shape[0], 1), lambda: (0, 0)),
                  pl.BlockSpec(clu_w.shape, lambda: (0, 0))],
        out_specs=[pl.BlockSpec((emb_w.shape[0], B), lambda: (0, 0)),
                   pl.BlockSpec((nc, B), lambda: (0, 0)),
                   pl.BlockSpec((nc, B), lambda: (0, 0)),
                   pl.BlockSpec((1, 8), lambda: (0, 0))],
        out_shape=[jax.ShapeDtypeStruct((emb_w.shape[0], B), F32),
                   jax.ShapeDtypeStruct((nc, B), F32),
                   jax.ShapeDtypeStruct((nc, B), F32),
                   jax.ShapeDtypeStruct((1, 8), F32)],
    )(xT, emb_w, emb_b[:, None], clu_w)
    return embT, disT, qT, sim


@jax.jit
def kernel(img1, img2, f1, f2, params):
    p = params
    img = jnp.concatenate([img1, img2], axis=0)
    f = jnp.concatenate([f1, f2], axis=0)
    B = img.shape[0]

    x = _conv(img, p['conv1_w'], p['conv1_b'], 5, 2, 2, 0.01)
    x = _conv(x, p['conv2_w'], p['conv2_b'], 5, 2, 2, 0.01)
    x = _conv(x, p['conv3_w'], p['conv3_b'], 3, 2, 0, 0.01)
    xT = x.reshape(B, -1).T

    g1 = _edgeconv(f, p['c1g_w'], p['bn1_g'], p['bn1_b'])
    g2 = _edgeconv(g1, p['c2g_w'], p['bn2_g'], p['bn2_b'])
    g3 = _edgeconv(g2, p['c3g_w'], p['bn3_g'], p['bn3_b'])
    g4 = _edgeconv(g3, p['c4g_w'], p['bn4_g'], p['bn4_b'])
    gcat = jnp.concatenate([g1, g2, g3, g4], axis=1)
    gm = _c5g_pool(gcat, p['c5g_w'], p['bn5_g'], p['bn5_b'])

    gT = _lin_T(p['lin_w'], gm.T, p['lin_b'], 1152)
    catT = jnp.concatenate([xT, gT], axis=0)
    hT = _lin1_T(p['lin1_w'], catT, p['lin1_b'], 2304)

    embT, disT, qT, sim = _head(hT, p['emb_w'], p['emb_b'], p['clu_w'])
    emb = embT.T
    dis = disT.T
    q = qT.T
    return (sim[0], q[0:8], q[8:16], emb[0:8], emb[8:16], dis[0:8])


# speedup vs baseline: 2.2306x; 1.0034x over previous
import functools

import jax
import jax.numpy as jnp
from jax import lax
from jax.experimental import pallas as pl
from jax.experimental.pallas import tpu as pltpu

N = 1024
KNN = 3
NEG = -3.0e38
HI = lax.Precision.HIGHEST
F32 = jnp.float32


def _mm_bias_lrelu_k(w_ref, b_ref, p_ref, o_ref, *, slope):
    y = jnp.dot(w_ref[...], p_ref[0], preferred_element_type=F32, precision=HI) + b_ref[...]
    o_ref[0] = jnp.where(y >= 0, y, slope * y)


def _im2col(x, ksize, stride, pad):
    B, C, H, W = x.shape
    if pad:
        x = jnp.pad(x, ((0, 0), (0, 0), (pad, pad), (pad, pad)))
    Ho = (H + 2 * pad - ksize) // stride + 1
    Wo = (W + 2 * pad - ksize) // stride + 1
    cols = []
    for di in range(ksize):
        for dj in range(ksize):
            cols.append(x[:, :, di:di + stride * (Ho - 1) + 1:stride,
                          dj:dj + stride * (Wo - 1) + 1:stride])
    p = jnp.stack(cols, axis=1)
    return p.reshape(B, ksize * ksize * C, Ho * Wo)


def _conv(x, w, b, ksize, stride, pad, slope):
    B = x.shape[0]
    O = w.shape[0]
    Ho = (x.shape[2] + 2 * pad - ksize) // stride + 1
    Wo = (x.shape[3] + 2 * pad - ksize) // stride + 1
    p = _im2col(x, ksize, stride, pad)
    Kd, Np = p.shape[1], p.shape[2]
    wf = w.transpose(0, 2, 3, 1).reshape(O, Kd)
    out = pl.pallas_call(
        functools.partial(_mm_bias_lrelu_k, slope=slope),
        grid=(B,),
        in_specs=[pl.BlockSpec((O, Kd), lambda i: (0, 0)),
                  pl.BlockSpec((O, 1), lambda i: (0, 0)),
                  pl.BlockSpec((1, Kd, Np), lambda i: (i, 0, 0))],
        out_specs=pl.BlockSpec((1, O, Np), lambda i: (i, 0, 0)),
        out_shape=jax.ShapeDtypeStruct((B, O, Np), F32),
        compiler_params=pltpu.CompilerParams(dimension_semantics=("parallel",)),
    )(wf, b[:, None], p)
    return out.reshape(B, O, Ho, Wo)


def _ec_k(x_ref, w1_ref, wd_ref, m_ref, s1_ref, s2_ref):
    i = pl.program_id(1)
    x = x_ref[0]
    w1 = w1_ref[...]
    wd = wd_ref[...]
    xtx = lax.dot_general(x, x, (((0,), (0,)), ((), ())),
                          preferred_element_type=F32, precision=HI)
    xx = jnp.sum(x * x, axis=0, keepdims=True)
    s = 2.0 * xtx - xx
    iota = lax.broadcasted_iota(jnp.int32, (N, N), 1)
    yc = jnp.dot(wd, x, preferred_element_type=F32, precision=HI)
    acc_max = acc_s1 = acc_s2 = None
    for k in range(KNN):
        v = jnp.max(s, axis=1, keepdims=True)
        t = jnp.where(s == v, iota, N)
        idx = jnp.min(t, axis=1, keepdims=True)
        sel = iota == idx
        oh = sel.astype(F32)
        xg = lax.dot_general(x, oh, (((1,), (1,)), ((), ())),
                             preferred_element_type=F32, precision=HI)
        out = jnp.dot(w1, xg, preferred_element_type=F32, precision=HI) + yc
        if k == 0:
            acc_max, acc_s1, acc_s2 = out, out, out * out
        else:
            acc_max = jnp.maximum(acc_max, out)
            acc_s1 = acc_s1 + out
            acc_s2 = acc_s2 + out * out
        if k < KNN - 1:
            s = jnp.where(sel, NEG, s)
    m_ref[0] = acc_max
    sum1 = jnp.sum(acc_s1, axis=1, keepdims=True)
    sum2 = jnp.sum(acc_s2, axis=1, keepdims=True)

    @pl.when(i == 0)
    def _():
        s1_ref[0] = sum1
        s2_ref[0] = sum2

    @pl.when(i != 0)
    def _():
        s1_ref[0] += sum1
        s2_ref[0] += sum2


def _aff_k(x_ref, sc_ref, sh_ref, o_ref, *, slope):
    z = x_ref[0] * sc_ref[0] + sh_ref[0]
    o_ref[0] = jnp.where(z >= 0, z, slope * z)


def _affine_lrelu(x, scale, shift, slope):
    B, O, n = x.shape
    half = B // 2
    return pl.pallas_call(
        functools.partial(_aff_k, slope=slope),
        grid=(B,),
        in_specs=[pl.BlockSpec((1, O, n), lambda b: (b, 0, 0)),
                  pl.BlockSpec((1, O, 1), lambda b: (b // half, 0, 0)),
                  pl.BlockSpec((1, O, 1), lambda b: (b // half, 0, 0))],
        out_specs=pl.BlockSpec((1, O, n), lambda b: (b, 0, 0)),
        out_shape=jax.ShapeDtypeStruct((B, O, n), F32),
        compiler_params=pltpu.CompilerParams(dimension_semantics=("parallel",)),
    )(x, scale, shift)


def _bn_affine(s1, s2, cnt, gamma, beta):
    mean = s1 / cnt
    var = s2 / cnt - mean * mean
    scale = gamma[None, :, None] / jnp.sqrt(var + 1e-5)
    shift = beta[None, :, None] - mean * scale
    return scale, shift


def _edgeconv(x, w, gamma, beta):
    B, C, n = x.shape
    O = w.shape[0]
    half = B // 2
    w1 = w[:, :C]
    wd = w[:, C:] - w1
    m, s1, s2 = pl.pallas_call(
        _ec_k,
        grid=(2, half),
        in_specs=[pl.BlockSpec((1, C, n), lambda h, i: (h * half + i, 0, 0)),
                  pl.BlockSpec((O, C), lambda h, i: (0, 0)),
                  pl.BlockSpec((O, C), lambda h, i: (0, 0))],
        out_specs=[pl.BlockSpec((1, O, n), lambda h, i: (h * half + i, 0, 0)),
                   pl.BlockSpec((1, O, 1), lambda h, i: (h, 0, 0)),
                   pl.BlockSpec((1, O, 1), lambda h, i: (h, 0, 0))],
        out_shape=[jax.ShapeDtypeStruct((B, O, n), F32),
                   jax.ShapeDtypeStruct((2, O, 1), F32),
                   jax.ShapeDtypeStruct((2, O, 1), F32)],
        compiler_params=pltpu.CompilerParams(
            dimension_semantics=("parallel", "arbitrary")),
    )(x, w1, wd)
    scale, shift = _bn_affine(s1, s2, half * n * KNN, gamma, beta)
    return _affine_lrelu(m, scale, shift, 0.2)


def _c5g_k(g_ref, w_ref, y_ref, s1_ref, s2_ref):
    i = pl.program_id(1)
    y = jnp.dot(w_ref[...], g_ref[0], preferred_element_type=F32, precision=HI)
    y_ref[0] = y
    sum1 = jnp.sum(y, axis=1, keepdims=True)
    sum2 = jnp.sum(y * y, axis=1, keepdims=True)

    @pl.when(i == 0)
    def _():
        s1_ref[0] = sum1
        s2_ref[0] = sum2

    @pl.when(i != 0)
    def _():
        s1_ref[0] += sum1
        s2_ref[0] += sum2


def _affred_k(y_ref, sc_ref, sh_ref, mx_ref, av_ref):
    z = y_ref[0] * sc_ref[0] + sh_ref[0]
    z = jnp.where(z >= 0, z, 0.2 * z)
    mx_ref[0] = jnp.max(z, axis=1, keepdims=True)
    av_ref[0] = jnp.sum(z, axis=1, keepdims=True) * (1.0 / N)


def _c5g_pool(g, w, gamma, beta):
    B, C, n = g.shape
    O = w.shape[0]
    half = B // 2
    y, s1, s2 = pl.pallas_call(
        _c5g_k,
        grid=(2, half),
        in_specs=[pl.BlockSpec((1, C, n), lambda h, i: (h * half + i, 0, 0)),
                  pl.BlockSpec((O, C), lambda h, i: (0, 0))],
        out_specs=[pl.BlockSpec((1, O, n), lambda h, i: (h * half + i, 0, 0)),
                   pl.BlockSpec((1, O, 1), lambda h, i: (h, 0, 0)),
                   pl.BlockSpec((1, O, 1), lambda h, i: (h, 0, 0))],
        out_shape=[jax.ShapeDtypeStruct((B, O, n), F32),
                   jax.ShapeDtypeStruct((2, O, 1), F32),
                   jax.ShapeDtypeStruct((2, O, 1), F32)],
        compiler_params=pltpu.CompilerParams(
            dimension_semantics=("parallel", "arbitrary")),
    )(g, w)
    scale, shift = _bn_affine(s1, s2, half * n, gamma, beta)
    gmax, gavg = pl.pallas_call(
        _affred_k,
        grid=(B,),
        in_specs=[pl.BlockSpec((1, O, n), lambda b: (b, 0, 0)),
                  pl.BlockSpec((1, O, 1), lambda b: (b // half, 0, 0)),
                  pl.BlockSpec((1, O, 1), lambda b: (b // half, 0, 0))],
        out_specs=[pl.BlockSpec((1, O, 1), lambda b: (b, 0, 0)),
                   pl.BlockSpec((1, O, 1), lambda b: (b, 0, 0))],
        out_shape=[jax.ShapeDtypeStruct((B, O, 1), F32),
                   jax.ShapeDtypeStruct((B, O, 1), F32)],
        compiler_params=pltpu.CompilerParams(dimension_semantics=("parallel",)),
    )(y, scale, shift)
    return jnp.concatenate([gmax[:, :, 0], gavg[:, :, 0]], axis=1)


def _linT_k(w_ref, a_ref, b_ref, o_ref):
    o_ref[...] = (jnp.dot(w_ref[...], a_ref[...], preferred_element_type=F32, precision=HI)
                  + b_ref[...])


def _lin_T(w, aT, b, row_tile):
    R, K = w.shape
    B = aT.shape[1]
    nt = R // row_tile
    return pl.pallas_call(
        _linT_k,
        grid=(nt,),
        in_specs=[pl.BlockSpec((row_tile, K), lambda t: (t, 0)),
                  pl.BlockSpec((K, B), lambda t: (0, 0)),
                  pl.BlockSpec((row_tile, 1), lambda t: (t, 0))],
        out_specs=pl.BlockSpec((row_tile, B), lambda t: (t, 0)),
        out_shape=jax.ShapeDtypeStruct((R, B), F32),
        compiler_params=pltpu.CompilerParams(dimension_semantics=("parallel",)),
    )(w, aT, b[:, None])


def _lin1_k(w_ref, a_ref, b_ref, o_ref):
    k = pl.program_id(1)
    part = jnp.dot(w_ref[...], a_ref[...], preferred_element_type=F32, precision=HI)

    @pl.when(k == 0)
    def _():
        o_ref[...] = part + b_ref[...]

    @pl.when(k != 0)
    def _():
        o_ref[...] += part


def _lin1_T(w, aT, b, k_tile):
    R, K = w.shape
    B = aT.shape[1]
    nk = K // k_tile
    rt = R // 2
    return pl.pallas_call(
        _lin1_k,
        grid=(2, nk),
        in_specs=[pl.BlockSpec((rt, k_tile), lambda r, k: (r, k)),
                  pl.BlockSpec((k_tile, B), lambda r, k: (k, 0)),
                  pl.BlockSpec((rt, 1), lambda r, k: (r, 0))],
        out_specs=pl.BlockSpec((rt, B), lambda r, k: (r, 0)),
        out_shape=jax.ShapeDtypeStruct((R, B), F32),
        compiler_params=pltpu.CompilerParams(
            dimension_semantics=("parallel", "arbitrary")),
    )(w, aT, b[:, None])


def _head_k(x_ref, ew_ref, eb_ref, cl_ref, emb_ref, dis_ref, q_ref, sim_ref):
    xT = x_ref[...]
    embT = (jnp.dot(ew_ref[...], xT, preferred_element_type=F32, precision=HI)
            + eb_ref[...])
    clu = cl_ref[...]
    cc = jnp.sum(clu * clu, axis=1, keepdims=True)
    ee = jnp.sum(embT * embT, axis=0, keepdims=True)
    cross = jnp.dot(clu, embT, preferred_element_type=F32, precision=HI)
    dis = cc - 2.0 * cross + ee
    q = 1.0 / (1.0 + dis)
    q = q / jnp.sum(q, axis=0, keepdims=True)
    d = embT[:, 0:8] - embT[:, 8:16] + 1e-6
    sim = jnp.sqrt(jnp.sum(d * d, axis=0, keepdims=True))
    emb_ref[...] = embT
    dis_ref[...] = dis
    q_ref[...] = q
    sim_ref[...] = sim


def _head(xT, emb_w, emb_b, clu_w):
    B = xT.shape[1]
    nc = clu_w.shape[0]
    embT, disT, qT, sim = pl.pallas_call(
        _head_k,
        in_specs=[pl.BlockSpec(xT.shape, lambda: (0, 0)),
                  pl.BlockSpec(emb_w.shape, lambda: (0, 0)),
                  pl.BlockSpec((emb_w.shape[0], 1), lambda: (0, 0)),
                  pl.BlockSpec(clu_w.shape, lambda: (0, 0))],
        out_specs=[pl.BlockSpec((emb_w.shape[0], B), lambda: (0, 0)),
                   pl.BlockSpec((nc, B), lambda: (0, 0)),
                   pl.BlockSpec((nc, B), lambda: (0, 0)),
                   pl.BlockSpec((1, 8), lambda: (0, 0))],
        out_shape=[jax.ShapeDtypeStruct((emb_w.shape[0], B), F32),
                   jax.ShapeDtypeStruct((nc, B), F32),
                   jax.ShapeDtypeStruct((nc, B), F32),
                   jax.ShapeDtypeStruct((1, 8), F32)],
    )(xT, emb_w, emb_b[:, None], clu_w)
    return embT, disT, qT, sim


@jax.jit
def kernel(img1, img2, f1, f2, params):
    p = params
    img = jnp.concatenate([img1, img2], axis=0)
    f = jnp.concatenate([f1, f2], axis=0)
    B = img.shape[0]

    x = _conv(img, p['conv1_w'], p['conv1_b'], 5, 2, 2, 0.01)
    x = _conv(x, p['conv2_w'], p['conv2_b'], 5, 2, 2, 0.01)
    x = _conv(x, p['conv3_w'], p['conv3_b'], 3, 2, 0, 0.01)
    xT = x.reshape(B, -1).T

    g1 = _edgeconv(f, p['c1g_w'], p['bn1_g'], p['bn1_b'])
    g2 = _edgeconv(g1, p['c2g_w'], p['bn2_g'], p['bn2_b'])
    g3 = _edgeconv(g2, p['c3g_w'], p['bn3_g'], p['bn3_b'])
    g4 = _edgeconv(g3, p['c4g_w'], p['bn4_g'], p['bn4_b'])
    gcat = jnp.concatenate([g1, g2, g3, g4], axis=1)
    gm = _c5g_pool(gcat, p['c5g_w'], p['bn5_g'], p['bn5_b'])

    gT = _lin_T(p['lin_w'], gm.T, p['lin_b'], 1152)
    catT = jnp.concatenate([xT, gT], axis=0)
    hT = _lin1_T(p['lin1_w'], catT, p['lin1_b'], 2304)

    embT, disT, qT, sim = _head(hT, p['emb_w'], p['emb_b'], p['clu_w'])
    emb = embT.T
    dis = disT.T
    q = qT.T
    return (sim[0], q[0:8], q[8:16], emb[0:8], emb[8:16], dis[0:8])


# speedup vs baseline: 5.3579x; 2.4020x over previous
import functools

import jax
import jax.numpy as jnp
from jax import lax
from jax.experimental import pallas as pl
from jax.experimental.pallas import tpu as pltpu

N = 1024
KNN = 3
NEG = -3.0e38
HI = lax.Precision.HIGHEST
F32 = jnp.float32


def _lrelu(y, slope):
    return jnp.where(y >= 0, y, slope * y)


def _pad_phases(ph):
    C = ph.shape[0]
    zr = jnp.zeros((C, 1, ph.shape[2]), F32)
    ph = jnp.concatenate([zr, ph, zr], axis=1)
    zc = jnp.zeros((C, ph.shape[1], 1), F32)
    return jnp.concatenate([zc, ph, zc], axis=2)


def _shift_patches(ph, nshift, tile):
    pats = []
    for a in range(nshift):
        for b in range(nshift):
            pats.append(ph[:, a:a + tile, b:b + tile]
                        .reshape(ph.shape[0], tile * tile))
    return jnp.concatenate(pats, axis=0)


def _conv1_k(x_ref, w_ref, b_ref, o_ref):
    ir = lax.broadcasted_iota(jnp.int32, (128, 128), 0)
    ic = lax.broadcasted_iota(jnp.int32, (128, 128), 1)
    S = (ir == 8 * (ic % 16) + ic // 16).astype(F32)
    phases = []
    for r in range(8):
        rp = x_ref[0, :, pl.ds(r, 16, 8), :]
        rp2 = jnp.dot(rp.reshape(48, 128), S, preferred_element_type=F32,
                      precision=HI).reshape(3, 16, 128)
        for s in range(8):
            phases.append(rp2[:, :, s * 16:(s + 1) * 16])
    ph = jnp.concatenate(phases, axis=0)
    P = _shift_patches(_pad_phases(ph), 3, 16)
    for op in range(16):
        y = jnp.dot(w_ref[op], P, preferred_element_type=F32,
                    precision=HI) + b_ref[...]
        o_ref[0, op * 32:(op + 1) * 32] = _lrelu(y, 0.01)


def _conv2_k(x_ref, w_ref, b_ref, o_ref):
    ph = x_ref[0].reshape(512, 16, 16)
    P = _shift_patches(_pad_phases(ph), 3, 16)
    for op in range(4):
        y = jnp.dot(w_ref[op], P, preferred_element_type=F32,
                    precision=HI) + b_ref[...]
        o_ref[0, op * 64:(op + 1) * 64] = _lrelu(y, 0.01)


def _conv3_k(x_ref, w_ref, b_ref, o_ref):
    ph = x_ref[0].reshape(256, 16, 16)
    pats = []
    for dr in range(2):
        for dc in range(2):
            pats.append(ph[:, dr:dr + 15, dc:dc + 15].reshape(256, 225))
    P = jnp.concatenate(pats, axis=0)
    y = jnp.dot(w_ref[...], P, preferred_element_type=F32,
                precision=HI) + b_ref[...]
    o_ref[0] = _lrelu(y, 0.01)


def _np_scatter_maps():
    import numpy as np
    M1 = np.zeros((16, 25, 9, 64), np.float32)
    for p in range(4):
        for q in range(4):
            for di in range(5):
                for dj in range(5):
                    ur, uc = 2 * p + di - 2, 2 * q + dj - 2
                    M1[p * 4 + q, di * 5 + dj,
                       (ur // 8 + 1) * 3 + (uc // 8 + 1),
                       (ur % 8) * 8 + (uc % 8)] = 1.0
    M2 = np.zeros((4, 25, 9, 16), np.float32)
    for s in range(2):
        for t in range(2):
            for di in range(5):
                for dj in range(5):
                    ur, uc = 2 * s + di - 2, 2 * t + dj - 2
                    M2[s * 2 + t, di * 5 + dj,
                       (ur // 4 + 1) * 3 + (uc // 4 + 1),
                       (ur % 4) * 4 + (uc % 4)] = 1.0
    M3 = np.zeros((1, 9, 4, 4), np.float32)
    for di in range(3):
        for dj in range(3):
            M3[0, di * 3 + dj, (di // 2) * 2 + (dj // 2),
               (di % 2) * 2 + (dj % 2)] = 1.0
    return (M1.reshape(16, 25, 576), M2.reshape(4, 25, 144),
            M3.reshape(1, 9, 16))


_M1, _M2, _M3 = _np_scatter_maps()


def _conv_call(kfn, x, ws, b, out_ch, out_sp):
    B = x.shape[0]
    return pl.pallas_call(
        kfn,
        grid=(B,),
        in_specs=[pl.BlockSpec((1,) + x.shape[1:], lambda i: (i,) + (0,) * (x.ndim - 1)),
                  pl.BlockSpec(ws.shape, lambda i: (0,) * ws.ndim),
                  pl.BlockSpec((b.shape[0], 1), lambda i: (0, 0))],
        out_specs=pl.BlockSpec((1, out_ch, out_sp), lambda i: (i, 0, 0)),
        out_shape=jax.ShapeDtypeStruct((B, out_ch, out_sp), F32),
        compiler_params=pltpu.CompilerParams(dimension_semantics=("parallel",)),
    )(x, ws, b[:, None])


def _conv_stack(img, p):
    B = img.shape[0]
    w1s = jnp.einsum('pkz,ock->pozc', _M1,
                     p['conv1_w'].reshape(32, 3, 25)).reshape(16, 32, 1728)
    w2s = jnp.einsum('pkz,ock->pozc', _M2,
                     p['conv2_w'].reshape(64, 32, 25)).reshape(4, 64, 4608)
    w3s = jnp.einsum('pkz,ock->pozc', _M3,
                     p['conv3_w'].reshape(128, 64, 9)).reshape(128, 1024)
    x1 = _conv_call(_conv1_k, img, w1s, p['conv1_b'], 512, 256)
    x2 = _conv_call(_conv2_k, x1.reshape(B, 512, 16, 16), w2s,
                    p['conv2_b'], 256, 256)
    x3 = _conv_call(_conv3_k, x2.reshape(B, 256, 16, 16), w3s,
                    p['conv3_b'], 128, 225)
    return x3.reshape(B, 28800)


def _ec_k(x_ref, w1_ref, wd_ref, m_ref, s1_ref, s2_ref):
    i = pl.program_id(1)
    x = x_ref[0]
    w1 = w1_ref[...]
    wd = wd_ref[...]
    xtx = lax.dot_general(x, x, (((0,), (0,)), ((), ())),
                          preferred_element_type=F32, precision=HI)
    xx = jnp.sum(x * x, axis=0, keepdims=True)
    s = 2.0 * xtx - xx
    iota = lax.broadcasted_iota(jnp.int32, (N, N), 1)
    yc = jnp.dot(wd, x, preferred_element_type=F32, precision=HI)
    acc_max = acc_s1 = acc_s2 = None
    for k in range(KNN):
        v = jnp.max(s, axis=1, keepdims=True)
        t = jnp.where(s == v, iota, N)
        idx = jnp.min(t, axis=1, keepdims=True)
        sel = iota == idx
        oh = sel.astype(F32)
        xg = lax.dot_general(x, oh, (((1,), (1,)), ((), ())),
                             preferred_element_type=F32, precision=HI)
        out = jnp.dot(w1, xg, preferred_element_type=F32, precision=HI) + yc
        if k == 0:
            acc_max, acc_s1, acc_s2 = out, out, out * out
        else:
            acc_max = jnp.maximum(acc_max, out)
            acc_s1 = acc_s1 + out
            acc_s2 = acc_s2 + out * out
        if k < KNN - 1:
            s = jnp.where(sel, NEG, s)
    m_ref[0] = acc_max
    sum1 = jnp.sum(acc_s1, axis=1, keepdims=True)
    sum2 = jnp.sum(acc_s2, axis=1, keepdims=True)

    @pl.when(i == 0)
    def _():
        s1_ref[0] = sum1
        s2_ref[0] = sum2

    @pl.when(i != 0)
    def _():
        s1_ref[0] += sum1
        s2_ref[0] += sum2


def _aff_k(x_ref, sc_ref, sh_ref, o_ref, *, slope):
    z = x_ref[0] * sc_ref[0] + sh_ref[0]
    o_ref[0] = jnp.where(z >= 0, z, slope * z)


def _affine_lrelu(x, scale, shift, slope):
    B, O, n = x.shape
    half = B // 2
    return pl.pallas_call(
        functools.partial(_aff_k, slope=slope),
        grid=(B,),
        in_specs=[pl.BlockSpec((1, O, n), lambda b: (b, 0, 0)),
                  pl.BlockSpec((1, O, 1), lambda b: (b // half, 0, 0)),
                  pl.BlockSpec((1, O, 1), lambda b: (b // half, 0, 0))],
        out_specs=pl.BlockSpec((1, O, n), lambda b: (b, 0, 0)),
        out_shape=jax.ShapeDtypeStruct((B, O, n), F32),
        compiler_params=pltpu.CompilerParams(dimension_semantics=("parallel",)),
    )(x, scale, shift)


def _bn_affine(s1, s2, cnt, gamma, beta):
    mean = s1 / cnt
    var = s2 / cnt - mean * mean
    scale = gamma[None, :, None] / jnp.sqrt(var + 1e-5)
    shift = beta[None, :, None] - mean * scale
    return scale, shift


def _edgeconv(x, w, gamma, beta):
    B, C, n = x.shape
    O = w.shape[0]
    half = B // 2
    w1 = w[:, :C]
    wd = w[:, C:] - w1
    m, s1, s2 = pl.pallas_call(
        _ec_k,
        grid=(2, half),
        in_specs=[pl.BlockSpec((1, C, n), lambda h, i: (h * half + i, 0, 0)),
                  pl.BlockSpec((O, C), lambda h, i: (0, 0)),
                  pl.BlockSpec((O, C), lambda h, i: (0, 0))],
        out_specs=[pl.BlockSpec((1, O, n), lambda h, i: (h * half + i, 0, 0)),
                   pl.BlockSpec((1, O, 1), lambda h, i: (h, 0, 0)),
                   pl.BlockSpec((1, O, 1), lambda h, i: (h, 0, 0))],
        out_shape=[jax.ShapeDtypeStruct((B, O, n), F32),
                   jax.ShapeDtypeStruct((2, O, 1), F32),
                   jax.ShapeDtypeStruct((2, O, 1), F32)],
        compiler_params=pltpu.CompilerParams(
            dimension_semantics=("parallel", "arbitrary")),
    )(x, w1, wd)
    scale, shift = _bn_affine(s1, s2, half * n * KNN, gamma, beta)
    return _affine_lrelu(m, scale, shift, 0.2)


def _c5g_k(g_ref, w_ref, y_ref, s1_ref, s2_ref):
    i = pl.program_id(1)
    y = jnp.dot(w_ref[...], g_ref[0], preferred_element_type=F32, precision=HI)
    y_ref[0] = y
    sum1 = jnp.sum(y, axis=1, keepdims=True)
    sum2 = jnp.sum(y * y, axis=1, keepdims=True)

    @pl.when(i == 0)
    def _():
        s1_ref[0] = sum1
        s2_ref[0] = sum2

    @pl.when(i != 0)
    def _():
        s1_ref[0] += sum1
        s2_ref[0] += sum2


def _affred_k(y_ref, sc_ref, sh_ref, mx_ref, av_ref):
    z = y_ref[0] * sc_ref[0] + sh_ref[0]
    z = jnp.where(z >= 0, z, 0.2 * z)
    mx_ref[0] = jnp.max(z, axis=1, keepdims=True)
    av_ref[0] = jnp.sum(z, axis=1, keepdims=True) * (1.0 / N)


def _c5g_pool(g, w, gamma, beta):
    B, C, n = g.shape
    O = w.shape[0]
    half = B // 2
    y, s1, s2 = pl.pallas_call(
        _c5g_k,
        grid=(2, half),
        in_specs=[pl.BlockSpec((1, C, n), lambda h, i: (h * half + i, 0, 0)),
                  pl.BlockSpec((O, C), lambda h, i: (0, 0))],
        out_specs=[pl.BlockSpec((1, O, n), lambda h, i: (h * half + i, 0, 0)),
                   pl.BlockSpec((1, O, 1), lambda h, i: (h, 0, 0)),
                   pl.BlockSpec((1, O, 1), lambda h, i: (h, 0, 0))],
        out_shape=[jax.ShapeDtypeStruct((B, O, n), F32),
                   jax.ShapeDtypeStruct((2, O, 1), F32),
                   jax.ShapeDtypeStruct((2, O, 1), F32)],
        compiler_params=pltpu.CompilerParams(
            dimension_semantics=("parallel", "arbitrary")),
    )(g, w)
    scale, shift = _bn_affine(s1, s2, half * n, gamma, beta)
    gmax, gavg = pl.pallas_call(
        _affred_k,
        grid=(B,),
        in_specs=[pl.BlockSpec((1, O, n), lambda b: (b, 0, 0)),
                  pl.BlockSpec((1, O, 1), lambda b: (b // half, 0, 0)),
                  pl.BlockSpec((1, O, 1), lambda b: (b // half, 0, 0))],
        out_specs=[pl.BlockSpec((1, O, 1), lambda b: (b, 0, 0)),
                   pl.BlockSpec((1, O, 1), lambda b: (b, 0, 0))],
        out_shape=[jax.ShapeDtypeStruct((B, O, 1), F32),
                   jax.ShapeDtypeStruct((B, O, 1), F32)],
        compiler_params=pltpu.CompilerParams(dimension_semantics=("parallel",)),
    )(y, scale, shift)
    return jnp.concatenate([gmax[:, :, 0], gavg[:, :, 0]], axis=1)


def _linT_k(w_ref, a_ref, b_ref, o_ref):
    o_ref[...] = (jnp.dot(w_ref[...], a_ref[...], preferred_element_type=F32, precision=HI)
                  + b_ref[...])


def _lin_T(w, aT, b, row_tile):
    R, K = w.shape
    B = aT.shape[1]
    nt = R // row_tile
    return pl.pallas_call(
        _linT_k,
        grid=(nt,),
        in_specs=[pl.BlockSpec((row_tile, K), lambda t: (t, 0)),
                  pl.BlockSpec((K, B), lambda t: (0, 0)),
                  pl.BlockSpec((row_tile, 1), lambda t: (t, 0))],
        out_specs=pl.BlockSpec((row_tile, B), lambda t: (t, 0)),
        out_shape=jax.ShapeDtypeStruct((R, B), F32),
        compiler_params=pltpu.CompilerParams(dimension_semantics=("parallel",)),
    )(w, aT, b[:, None])


def _lin1_k(w_ref, a_ref, b_ref, o_ref):
    k = pl.program_id(1)
    part = jnp.dot(w_ref[...], a_ref[...], preferred_element_type=F32, precision=HI)

    @pl.when(k == 0)
    def _():
        o_ref[...] = part + b_ref[...]

    @pl.when(k != 0)
    def _():
        o_ref[...] += part


def _lin1_T(w, aT, b, k_tile):
    R, K = w.shape
    B = aT.shape[1]
    nk = K // k_tile
    rt = R // 2
    return pl.pallas_call(
        _lin1_k,
        grid=(2, nk),
        in_specs=[pl.BlockSpec((rt, k_tile), lambda r, k: (r, k)),
                  pl.BlockSpec((k_tile, B), lambda r, k: (k, 0)),
                  pl.BlockSpec((rt, 1), lambda r, k: (r, 0))],
        out_specs=pl.BlockSpec((rt, B), lambda r, k: (r, 0)),
        out_shape=jax.ShapeDtypeStruct((R, B), F32),
        compiler_params=pltpu.CompilerParams(
            dimension_semantics=("parallel", "arbitrary")),
    )(w, aT, b[:, None])


def _head_k(x_ref, ew_ref, eb_ref, cl_ref, emb_ref, dis_ref, q_ref, sim_ref):
    xT = x_ref[...]
    embT = (jnp.dot(ew_ref[...], xT, preferred_element_type=F32, precision=HI)
            + eb_ref[...])
    clu = cl_ref[...]
    cc = jnp.sum(clu * clu, axis=1, keepdims=True)
    ee = jnp.sum(embT * embT, axis=0, keepdims=True)
    cross = jnp.dot(clu, embT, preferred_element_type=F32, precision=HI)
    dis = cc - 2.0 * cross + ee
    q = 1.0 / (1.0 + dis)
    q = q / jnp.sum(q, axis=0, keepdims=True)
    d = embT[:, 0:8] - embT[:, 8:16] + 1e-6
    sim = jnp.sqrt(jnp.sum(d * d, axis=0, keepdims=True))
    emb_ref[...] = embT
    dis_ref[...] = dis
    q_ref[...] = q
    sim_ref[...] = sim


def _head(xT, emb_w, emb_b, clu_w):
    B = xT.shape[1]
    nc = clu_w.shape[0]
    embT, disT, qT, sim = pl.pallas_call(
        _head_k,
        in_specs=[pl.BlockSpec(xT.shape, lambda: (0, 0)),
                  pl.BlockSpec(emb_w.shape, lambda: (0, 0)),
                  pl.BlockSpec((emb_w.shape[0], 1), lambda: (0, 0)),
                  pl.BlockSpec(clu_w.shape, lambda: (0, 0))],
        out_specs=[pl.BlockSpec((emb_w.shape[0], B), lambda: (0, 0)),
                   pl.BlockSpec((nc, B), lambda: (0, 0)),
                   pl.BlockSpec((nc, B), lambda: (0, 0)),
                   pl.BlockSpec((1, 8), lambda: (0, 0))],
        out_shape=[jax.ShapeDtypeStruct((emb_w.shape[0], B), F32),
                   jax.ShapeDtypeStruct((nc, B), F32),
                   jax.ShapeDtypeStruct((nc, B), F32),
                   jax.ShapeDtypeStruct((1, 8), F32)],
    )(xT, emb_w, emb_b[:, None], clu_w)
    return embT, disT, qT, sim


@jax.jit
def kernel(img1, img2, f1, f2, params):
    p = params
    img = jnp.concatenate([img1, img2], axis=0)
    f = jnp.concatenate([f1, f2], axis=0)
    B = img.shape[0]

    xT = _conv_stack(img, p).T

    g1 = _edgeconv(f, p['c1g_w'], p['bn1_g'], p['bn1_b'])
    g2 = _edgeconv(g1, p['c2g_w'], p['bn2_g'], p['bn2_b'])
    g3 = _edgeconv(g2, p['c3g_w'], p['bn3_g'], p['bn3_b'])
    g4 = _edgeconv(g3, p['c4g_w'], p['bn4_g'], p['bn4_b'])
    gcat = jnp.concatenate([g1, g2, g3, g4], axis=1)
    gm = _c5g_pool(gcat, p['c5g_w'], p['bn5_g'], p['bn5_b'])

    gT = _lin_T(p['lin_w'], gm.T, p['lin_b'], 1152)
    catT = jnp.concatenate([xT, gT], axis=0)
    hT = _lin1_T(p['lin1_w'], catT, p['lin1_b'], 2304)

    embT, disT, qT, sim = _head(hT, p['emb_w'], p['emb_b'], p['clu_w'])
    emb = embT.T
    dis = disT.T
    q = qT.T
    return (sim[0], q[0:8], q[8:16], emb[0:8], emb[8:16], dis[0:8])
